# baseline (speedup 1.0000x reference)
import dataclasses
import functools
import math

import jax
import jax.numpy as jnp
import numpy as np
from jax import lax
from jax.experimental import pallas as pl
from jax.experimental.pallas import tpu as pltpu
from jax.experimental.pallas import tpu_sc as plsc

B = 8
D_IN = 2048
KDIM = 8192
NACT = 81
BK = 512
GRID = KDIM // BK
CHUNK = KDIM // 4
L = 16

_INT_MIN = np.int32(-2147483648)
_MASK31 = np.int32(0x7FFFFFFF)


def _count_ge(key, thresh):
    return jnp.sum((key >= thresh).astype(jnp.int32), axis=1, keepdims=True)


def _tc_body(z_ref, pr_ref, pi_ref, phi_ref, lamr_ref, lami_ref,
             scores_ref, tcx_ref, pf_ref, cphi_ref, sphi_ref, cx_scr):
    i = pl.program_id(0)
    pr = pr_ref[...]
    pi = pi_ref[...]
    ss = jnp.sum(pr * pr + pi * pi, axis=1, keepdims=True)
    nrm = jnp.maximum(jnp.sqrt(ss), 1e-12)
    prn = pr / nrm
    s = lax.dot_general(z_ref[...], prn, (((1,), (1,)), ((), ())),
                        preferred_element_type=jnp.float32)
    scores_ref[:, pl.ds(pl.multiple_of(i * BK, BK), BK)] = s

    @pl.when(i == 0)
    def _trig():
        phi = phi_ref[...]
        cphi_ref[...] = jnp.cos(phi)
        sphi_ref[...] = jnp.sin(phi)

    @pl.when(i == GRID - 1)
    def _finish():
        scores = scores_ref[...]
        raw = lax.bitcast_convert_type(scores, jnp.int32)
        sgn = lax.shift_right_arithmetic(raw, 31)
        key = jnp.bitwise_xor(raw, jnp.bitwise_and(sgn, _MASK31))

        c = jnp.zeros((B, 1), jnp.int32)
        for b in range(31, -1, -1):
            bit = _INT_MIN if b == 31 else jnp.int32(1 << b)
            cand = jnp.bitwise_or(c, bit)
            cnt = _count_ge(key, jnp.bitwise_xor(cand, _INT_MIN))
            c = jnp.where(cnt >= NACT, cand, c)
        t_s = jnp.bitwise_xor(c, _INT_MIN)

        is_gt = key > t_s
        is_eq = key == t_s
        cnt_gt = jnp.sum(is_gt.astype(jnp.int32), axis=1, keepdims=True)
        cnt_eq = jnp.sum(is_eq.astype(jnp.int32), axis=1, keepdims=True)
        need = NACT - cnt_gt
        iota = lax.broadcasted_iota(jnp.int32, (B, KDIM), 1)
        cx_scr[...] = jnp.full((B, 1), KDIM - 1, jnp.int32)

        @pl.when(jnp.any(cnt_eq > need))
        def _ties():
            cx = jnp.zeros((B, 1), jnp.int32)
            for b in range(12, -1, -1):
                cand = jnp.bitwise_or(cx, jnp.int32(1 << b))
                cnt = jnp.sum((is_eq & (iota < cand)).astype(jnp.int32),
                              axis=1, keepdims=True)
                cx = jnp.where(cnt < need, cand, cx)
            cx_scr[...] = cx

        cx = cx_scr[...]
        mask = is_gt | (is_eq & (iota <= cx))
        sv = jnp.where(mask, scores, 0.0)

        lam_r = lamr_ref[0, 0]
        lam_i = lami_ref[0, 0]
        z = z_ref[...]
        cphi1 = cphi_ref[:, :D_IN]
        sphi1 = sphi_ref[:, :D_IN]
        sv1 = sv[:, :D_IN]
        n2 = (jnp.sum(sv * sv, axis=1, keepdims=True)
              + 2.0 * jnp.sum(sv1 * z * (lam_r * cphi1 + lam_i * sphi1),
                              axis=1, keepdims=True)
              + (lam_r * lam_r + lam_i * lam_i)
              * jnp.sum(z * z, axis=1, keepdims=True))
        scale = math.sqrt(KDIM) / jnp.maximum(jnp.sqrt(n2), 1e-12)

        tcx_ref[...] = jnp.concatenate(
            [jnp.broadcast_to(t_s, (B, 16)), jnp.broadcast_to(cx, (B, 16)),
             jnp.zeros((B, 96), jnp.int32)], axis=1)
        pf_ref[...] = jnp.concatenate(
            [jnp.broadcast_to(scale, (B, 16)),
             jnp.full((B, 16), lam_r), jnp.full((B, 16), lam_i),
             jnp.zeros((B, 80), jnp.float32)], axis=1)


def _tc_call(z_in, P_real, P_imag, phi2d, lam_r, lam_i):
    return pl.pallas_call(
        _tc_body,
        grid=(GRID,),
        in_specs=[
            pl.BlockSpec((B, D_IN), lambda i: (0, 0)),
            pl.BlockSpec((BK, D_IN), lambda i: (i, 0)),
            pl.BlockSpec((BK, D_IN), lambda i: (i, 0)),
            pl.BlockSpec((1, KDIM), lambda i: (0, 0)),
            pl.BlockSpec(memory_space=pltpu.SMEM),
            pl.BlockSpec(memory_space=pltpu.SMEM),
        ],
        out_specs=[
            pl.BlockSpec((B, KDIM), lambda i: (0, 0)),
            pl.BlockSpec((B, 128), lambda i: (0, 0)),
            pl.BlockSpec((B, 128), lambda i: (0, 0)),
            pl.BlockSpec((1, KDIM), lambda i: (0, 0)),
            pl.BlockSpec((1, KDIM), lambda i: (0, 0)),
        ],
        out_shape=[
            jax.ShapeDtypeStruct((B, KDIM), jnp.float32),
            jax.ShapeDtypeStruct((B, 128), jnp.int32),
            jax.ShapeDtypeStruct((B, 128), jnp.float32),
            jax.ShapeDtypeStruct((1, KDIM), jnp.float32),
            jax.ShapeDtypeStruct((1, KDIM), jnp.float32),
        ],
        scratch_shapes=[pltpu.VMEM((B, 1), jnp.int32)],
        compiler_params=pltpu.CompilerParams(
            dimension_semantics=("arbitrary",)),
    )(z_in, P_real, P_imag, phi2d, lam_r, lam_i)


def _sc_body(scores_hbm, tcx_hbm, pf_hbm, cphi_hbm, sphi_hbm, z_hbm,
             out_r_hbm, out_i_hbm,
             s_v, c_v, sn_v, z_v, tcx_v, pf_v, or_v, oi_v):
    cid = lax.axis_index("c")
    sid = lax.axis_index("s")
    r = jnp.bitwise_and(sid, 7)
    q = 2 * cid + lax.shift_right_logical(sid, 3)
    base = q * CHUNK

    pltpu.sync_copy(scores_hbm.at[r, pl.ds(base, CHUNK)], s_v)
    pltpu.sync_copy(cphi_hbm.at[pl.ds(base, CHUNK)], c_v)
    pltpu.sync_copy(sphi_hbm.at[pl.ds(base, CHUNK)], sn_v)
    pltpu.sync_copy(tcx_hbm.at[r], tcx_v)
    pltpu.sync_copy(pf_hbm.at[r], pf_v)

    t_b = tcx_v[pl.ds(0, L)]
    cx_b = tcx_v[pl.ds(16, L)]
    scale_b = pf_v[pl.ds(0, L)]
    lamr_b = pf_v[pl.ds(16, L)]
    lami_b = pf_v[pl.ds(32, L)]
    lane = lax.iota(jnp.int32, L)

    def _mask_sv(j):
        s = s_v[pl.ds(j, L)]
        raw = plsc.bitcast(s, jnp.int32)
        sgn = lax.shift_right_arithmetic(raw, 31)
        key = jnp.bitwise_xor(raw, jnp.bitwise_and(sgn, _MASK31))
        gidx = lane + (base + j)
        keep = (key > t_b) | ((key == t_b) & (gidx <= cx_b))
        return jnp.where(keep, s, 0.0)

    @pl.when(q == 0)
    def _first_quarter():
        pltpu.sync_copy(z_hbm.at[r], z_v)

        @pl.loop(0, CHUNK, step=L)
        def _(j):
            sv = _mask_sv(j)
            zc = z_v[pl.ds(j, L)]
            or_v[pl.ds(j, L)] = (sv * c_v[pl.ds(j, L)]
                                 + lamr_b * zc) * scale_b
            oi_v[pl.ds(j, L)] = (sv * sn_v[pl.ds(j, L)]
                                 + lami_b * zc) * scale_b

    @pl.when(q != 0)
    def _rest():
        @pl.loop(0, CHUNK, step=L)
        def _(j):
            sv = _mask_sv(j)
            or_v[pl.ds(j, L)] = sv * c_v[pl.ds(j, L)] * scale_b
            oi_v[pl.ds(j, L)] = sv * sn_v[pl.ds(j, L)] * scale_b

    pltpu.sync_copy(or_v, out_r_hbm.at[r, pl.ds(base, CHUNK)])
    pltpu.sync_copy(oi_v, out_i_hbm.at[r, pl.ds(base, CHUNK)])


def _sc_call(scores, tcx, pf, cphi, sphi, z_in):
    mesh = plsc.VectorSubcoreMesh(core_axis_name="c", subcore_axis_name="s")
    cp = pltpu.CompilerParams()
    if "needs_layout_passes" in pltpu.CompilerParams.__dataclass_fields__:
        cp = dataclasses.replace(cp, needs_layout_passes=False)
    run = pl.kernel(
        _sc_body,
        out_type=[
            jax.ShapeDtypeStruct((B, KDIM), jnp.float32),
            jax.ShapeDtypeStruct((B, KDIM), jnp.float32),
        ],
        mesh=mesh,
        scratch_types=[
            pltpu.VMEM((CHUNK,), jnp.float32),
            pltpu.VMEM((CHUNK,), jnp.float32),
            pltpu.VMEM((CHUNK,), jnp.float32),
            pltpu.VMEM((D_IN,), jnp.float32),
            pltpu.VMEM((128,), jnp.int32),
            pltpu.VMEM((128,), jnp.float32),
            pltpu.VMEM((CHUNK,), jnp.float32),
            pltpu.VMEM((CHUNK,), jnp.float32),
        ],
        compiler_params=cp,
    )
    return run(scores, tcx, pf, cphi, sphi, z_in)


def kernel(z_in, P_real, P_imag, phi, lam_real, lam_imag):
    phi2d = phi.reshape(1, KDIM)
    lam_r = lam_real.reshape(1, 1).astype(jnp.float32)
    lam_i = lam_imag.reshape(1, 1).astype(jnp.float32)
    scores, tcx, pf, cphi, sphi = _tc_call(
        z_in, P_real, P_imag, phi2d, lam_r, lam_i)
    out_r, out_i = _sc_call(scores, tcx, pf,
                            cphi.reshape(KDIM), sphi.reshape(KDIM), z_in)
    return lax.complex(out_r, out_i)


# speedup vs baseline: 1.0311x; 1.0311x over previous
import dataclasses
import functools
import math

import jax
import jax.numpy as jnp
import numpy as np
from jax import lax
from jax.experimental import pallas as pl
from jax.experimental.pallas import tpu as pltpu
from jax.experimental.pallas import tpu_sc as plsc

B = 8
D_IN = 2048
KDIM = 8192
NACT = 81
BK = 512
GRID = KDIM // BK
CHUNK = KDIM // 4
L = 16

_INT_MIN = np.int32(-2147483648)
_MASK31 = np.int32(0x7FFFFFFF)


def _count_ge(key, thresh):
    return jnp.sum((key >= thresh).astype(jnp.int32), axis=1, keepdims=True)


def _tc_body(z_ref, pr_ref, pi_ref, phi_ref, lamr_ref, lami_ref,
             scores_ref, tcx_ref, pf_ref, cphi_ref, sphi_ref, cx_scr):
    i = pl.program_id(0)
    pr = pr_ref[...]
    pi = pi_ref[...]
    ss = jnp.sum(pr * pr + pi * pi, axis=1, keepdims=True)
    nrm = jnp.maximum(jnp.sqrt(ss), 1e-12)
    prn = pr / nrm
    s = lax.dot_general(z_ref[...], prn, (((1,), (1,)), ((), ())),
                        preferred_element_type=jnp.float32)
    scores_ref[:, pl.ds(pl.multiple_of(i * BK, BK), BK)] = s

    @pl.when(i == 0)
    def _trig():
        phi = phi_ref[...]
        cphi_ref[...] = jnp.cos(phi)
        sphi_ref[...] = jnp.sin(phi)

    @pl.when(i == GRID - 1)
    def _finish():
        scores = scores_ref[...]
        raw = lax.bitcast_convert_type(scores, jnp.int32)
        sgn = lax.shift_right_arithmetic(raw, 31)
        key = jnp.bitwise_xor(raw, jnp.bitwise_and(sgn, _MASK31))

        c = jnp.zeros((B, 1), jnp.int32)
        for b in range(31, -1, -1):
            bit = _INT_MIN if b == 31 else jnp.int32(1 << b)
            cand = jnp.bitwise_or(c, bit)
            cnt = _count_ge(key, jnp.bitwise_xor(cand, _INT_MIN))
            c = jnp.where(cnt >= NACT, cand, c)
        t_s = jnp.bitwise_xor(c, _INT_MIN)

        is_gt = key > t_s
        is_eq = key == t_s
        cnt_gt = jnp.sum(is_gt.astype(jnp.int32), axis=1, keepdims=True)
        cnt_eq = jnp.sum(is_eq.astype(jnp.int32), axis=1, keepdims=True)
        need = NACT - cnt_gt
        iota = lax.broadcasted_iota(jnp.int32, (B, KDIM), 1)
        cx_scr[...] = jnp.full((B, 1), KDIM - 1, jnp.int32)

        @pl.when(jnp.any(cnt_eq > need))
        def _ties():
            cx = jnp.zeros((B, 1), jnp.int32)
            for b in range(12, -1, -1):
                cand = jnp.bitwise_or(cx, jnp.int32(1 << b))
                cnt = jnp.sum((is_eq & (iota < cand)).astype(jnp.int32),
                              axis=1, keepdims=True)
                cx = jnp.where(cnt < need, cand, cx)
            cx_scr[...] = cx

        cx = cx_scr[...]
        mask = is_gt | (is_eq & (iota <= cx))
        sv = jnp.where(mask, scores, 0.0)

        lam_r = lamr_ref[0, 0]
        lam_i = lami_ref[0, 0]
        z = z_ref[...]
        cphi1 = cphi_ref[:, :D_IN]
        sphi1 = sphi_ref[:, :D_IN]
        sv1 = sv[:, :D_IN]
        n2 = (jnp.sum(sv * sv, axis=1, keepdims=True)
              + 2.0 * jnp.sum(sv1 * z * (lam_r * cphi1 + lam_i * sphi1),
                              axis=1, keepdims=True)
              + (lam_r * lam_r + lam_i * lam_i)
              * jnp.sum(z * z, axis=1, keepdims=True))
        scale = math.sqrt(KDIM) / jnp.maximum(jnp.sqrt(n2), 1e-12)

        tcx_ref[...] = jnp.concatenate(
            [jnp.broadcast_to(t_s, (B, 16)), jnp.broadcast_to(cx, (B, 16)),
             jnp.zeros((B, 96), jnp.int32)], axis=1)
        pf_ref[...] = jnp.concatenate(
            [jnp.broadcast_to(scale, (B, 16)),
             jnp.full((B, 16), lam_r), jnp.full((B, 16), lam_i),
             jnp.zeros((B, 80), jnp.float32)], axis=1)


def _tc_call(z_in, P_real, P_imag, phi2d, lam_r, lam_i):
    return pl.pallas_call(
        _tc_body,
        grid=(GRID,),
        in_specs=[
            pl.BlockSpec((B, D_IN), lambda i: (0, 0)),
            pl.BlockSpec((BK, D_IN), lambda i: (i, 0)),
            pl.BlockSpec((BK, D_IN), lambda i: (i, 0)),
            pl.BlockSpec((1, KDIM), lambda i: (0, 0)),
            pl.BlockSpec(memory_space=pltpu.SMEM),
            pl.BlockSpec(memory_space=pltpu.SMEM),
        ],
        out_specs=[
            pl.BlockSpec((B, KDIM), lambda i: (0, 0)),
            pl.BlockSpec((B, 128), lambda i: (0, 0)),
            pl.BlockSpec((B, 128), lambda i: (0, 0)),
            pl.BlockSpec((1, KDIM), lambda i: (0, 0)),
            pl.BlockSpec((1, KDIM), lambda i: (0, 0)),
        ],
        out_shape=[
            jax.ShapeDtypeStruct((B, KDIM), jnp.float32),
            jax.ShapeDtypeStruct((B, 128), jnp.int32),
            jax.ShapeDtypeStruct((B, 128), jnp.float32),
            jax.ShapeDtypeStruct((1, KDIM), jnp.float32),
            jax.ShapeDtypeStruct((1, KDIM), jnp.float32),
        ],
        scratch_shapes=[pltpu.VMEM((B, 1), jnp.int32)],
        compiler_params=pltpu.CompilerParams(
            dimension_semantics=("arbitrary",)),
    )(z_in, P_real, P_imag, phi2d, lam_r, lam_i)


def _sc_body(scores_hbm, tcx_hbm, pf_hbm, cphi_hbm, sphi_hbm, z_hbm,
             out_r_hbm, out_i_hbm,
             s_v, c_v, sn_v, z_v, tcx_v, pf_v, or_v, oi_v, sem):
    cid = lax.axis_index("c")
    sid = lax.axis_index("s")
    r = jnp.bitwise_and(sid, 7)
    q = 2 * cid + lax.shift_right_logical(sid, 3)
    base = q * CHUNK

    cps = [pltpu.async_copy(scores_hbm.at[r, pl.ds(base, CHUNK)], s_v, sem),
           pltpu.async_copy(cphi_hbm.at[pl.ds(base, CHUNK)], c_v, sem),
           pltpu.async_copy(sphi_hbm.at[pl.ds(base, CHUNK)], sn_v, sem),
           pltpu.async_copy(tcx_hbm.at[r], tcx_v, sem),
           pltpu.async_copy(pf_hbm.at[r], pf_v, sem),
           pltpu.async_copy(z_hbm.at[r], z_v, sem)]
    for cp in cps:
        cp.wait()

    t_b = tcx_v[pl.ds(0, L)]
    cx_b = tcx_v[pl.ds(16, L)]
    scale_b = pf_v[pl.ds(0, L)]
    lamr_b = pf_v[pl.ds(16, L)]
    lami_b = pf_v[pl.ds(32, L)]
    lane = lax.iota(jnp.int32, L)

    def _mask_sv(j):
        s = s_v[pl.ds(j, L)]
        raw = plsc.bitcast(s, jnp.int32)
        sgn = lax.shift_right_arithmetic(raw, 31)
        key = jnp.bitwise_xor(raw, jnp.bitwise_and(sgn, _MASK31))
        gidx = lane + (base + j)
        keep = (key > t_b) | ((key == t_b) & (gidx <= cx_b))
        return jnp.where(keep, s, 0.0)

    @pl.when(q == 0)
    def _first_quarter():
        @pl.loop(0, CHUNK, step=L)
        def _(j):
            sv = _mask_sv(j)
            zc = z_v[pl.ds(j, L)]
            or_v[pl.ds(j, L)] = (sv * c_v[pl.ds(j, L)]
                                 + lamr_b * zc) * scale_b
            oi_v[pl.ds(j, L)] = (sv * sn_v[pl.ds(j, L)]
                                 + lami_b * zc) * scale_b

    @pl.when(q != 0)
    def _rest():
        @pl.loop(0, CHUNK, step=L)
        def _(j):
            sv = _mask_sv(j)
            or_v[pl.ds(j, L)] = sv * c_v[pl.ds(j, L)] * scale_b
            oi_v[pl.ds(j, L)] = sv * sn_v[pl.ds(j, L)] * scale_b

    co1 = pltpu.async_copy(or_v, out_r_hbm.at[r, pl.ds(base, CHUNK)], sem)
    co2 = pltpu.async_copy(oi_v, out_i_hbm.at[r, pl.ds(base, CHUNK)], sem)
    co1.wait()
    co2.wait()


def _sc_call(scores, tcx, pf, cphi, sphi, z_in):
    mesh = plsc.VectorSubcoreMesh(core_axis_name="c", subcore_axis_name="s")
    cp = pltpu.CompilerParams()
    if "needs_layout_passes" in pltpu.CompilerParams.__dataclass_fields__:
        cp = dataclasses.replace(cp, needs_layout_passes=False)
    run = pl.kernel(
        _sc_body,
        out_type=[
            jax.ShapeDtypeStruct((B, KDIM), jnp.float32),
            jax.ShapeDtypeStruct((B, KDIM), jnp.float32),
        ],
        mesh=mesh,
        scratch_types=[
            pltpu.VMEM((CHUNK,), jnp.float32),
            pltpu.VMEM((CHUNK,), jnp.float32),
            pltpu.VMEM((CHUNK,), jnp.float32),
            pltpu.VMEM((D_IN,), jnp.float32),
            pltpu.VMEM((128,), jnp.int32),
            pltpu.VMEM((128,), jnp.float32),
            pltpu.VMEM((CHUNK,), jnp.float32),
            pltpu.VMEM((CHUNK,), jnp.float32),
            pltpu.SemaphoreType.DMA,
        ],
        compiler_params=cp,
    )
    return run(scores, tcx, pf, cphi, sphi, z_in)


def kernel(z_in, P_real, P_imag, phi, lam_real, lam_imag):
    phi2d = phi.reshape(1, KDIM)
    lam_r = lam_real.reshape(1, 1).astype(jnp.float32)
    lam_i = lam_imag.reshape(1, 1).astype(jnp.float32)
    scores, tcx, pf, cphi, sphi = _tc_call(
        z_in, P_real, P_imag, phi2d, lam_r, lam_i)
    out_r, out_i = _sc_call(scores, tcx, pf,
                            cphi.reshape(KDIM), sphi.reshape(KDIM), z_in)
    return lax.complex(out_r, out_i)
